# baseline (speedup 1.0000x reference)
import functools

import jax
import jax.numpy as jnp
from jax.experimental import pallas as pl
from jax.experimental.pallas import tpu as pltpu
from jax.experimental.pallas import tpu_sc as plsc

K_NN = 16
N_GRAPHS = 16

INF_KEY = 0x7F800000
DEAD_KEY = 0x7FFFFFFF


TILE_DST = 128
BLK_CAND = 320


def _knn_body(lo_ref, hi_ref, pcand_ref, bcand_ref, pdst_ref, bdst_ref,
              nbr_ref, k_ref, *, toff):
    t = pl.program_id(0)
    lo_b = lo_ref[t]
    hi_b = hi_ref[t]

    pd0 = pdst_ref[0:1, :]
    pd1 = pdst_ref[1:2, :]
    pd2 = pdst_ref[2:3, :]
    bd = bdst_ref[0:1, :]
    did = (t + toff) * TILE_DST + jax.lax.broadcasted_iota(
        jnp.int32, (1, TILE_DST), 1)

    big = jnp.int32(2 ** 30)

    def _top2(kb, cid, carry):
        r1m, r1i, r2m, r2i = carry
        m1 = jnp.min(kb, axis=0, keepdims=True)
        i1 = jnp.min(jnp.where(kb == m1, cid, big), axis=0, keepdims=True)
        kb2 = jnp.where(cid == i1, DEAD_KEY, kb)
        m2 = jnp.min(kb2, axis=0, keepdims=True)
        i2 = jnp.min(jnp.where(kb2 == m2, cid, big), axis=0, keepdims=True)
        lt = lambda am, ai, bm, bi: (am < bm) | ((am == bm) & (ai < bi))
        c = lt(m1, i1, r1m, r1i)
        n1m, n1i = jnp.where(c, m1, r1m), jnp.where(c, i1, r1i)
        h1m, h1i = jnp.where(c, r1m, m1), jnp.where(c, r1i, i1)
        d = lt(m2, i2, r2m, r2i)
        l2m, l2i = jnp.where(d, m2, r2m), jnp.where(d, i2, r2i)
        e = lt(h1m, h1i, l2m, l2i)
        n2m, n2i = jnp.where(e, h1m, l2m), jnp.where(e, h1i, l2i)
        return (n1m, n1i, n2m, n2i)

    init2 = (jnp.full((1, TILE_DST), DEAD_KEY, jnp.int32),
             jnp.full((1, TILE_DST), big, jnp.int32),
             jnp.full((1, TILE_DST), DEAD_KEY, jnp.int32),
             jnp.full((1, TILE_DST), big, jnp.int32))

    def dist_block(b, carry):
        base = b * BLK_CAND
        sl = pl.ds(base, BLK_CAND)
        dx0 = pcand_ref[sl, 0:1] - pd0
        dx1 = pcand_ref[sl, 1:2] - pd1
        dx2 = pcand_ref[sl, 2:3] - pd2
        d = dx0 * dx0 + dx1 * dx1 + dx2 * dx2
        bc = bcand_ref[sl, 0:1]
        cid = base + jax.lax.broadcasted_iota(jnp.int32, (BLK_CAND, 1), 0)
        mask = (bc != bd) | (cid == did)
        key = jax.lax.bitcast_convert_type(d, jnp.int32)
        kv = jnp.where(mask, INF_KEY, key)
        k_ref[sl, :] = kv
        return _top2(kv, cid, carry)

    carry = jax.lax.fori_loop(lo_b, hi_b, dist_block, init2)
    picks = [carry[1], carry[3]]
    li1, li2 = carry[1], carry[3]

    for p in range(1, K_NN // 2):
        last = p == K_NN // 2 - 1

        def scan_block(b, carry, li1=li1, li2=li2, last=last):
            base = b * BLK_CAND
            sl = pl.ds(base, BLK_CAND)
            cid = base + jax.lax.broadcasted_iota(
                jnp.int32, (BLK_CAND, 1), 0)
            kb = k_ref[sl, :]
            kb = jnp.where((cid == li1) | (cid == li2), DEAD_KEY, kb)
            if not last:
                k_ref[sl, :] = kb
            return _top2(kb, cid, carry)

        carry = jax.lax.fori_loop(lo_b, hi_b, scan_block, init2)
        picks += [carry[1], carry[3]]
        li1, li2 = carry[1], carry[3]

    nbr_ref[...] = jnp.concatenate(picks, axis=0)


def _knn_pallas(pcand, bcand, pdst, bdst, lo_blk, hi_blk, npad, toff, nt):
    body = functools.partial(_knn_body, toff=toff)
    return pl.pallas_call(
        body,
        grid=(nt,),
        in_specs=[
            pl.BlockSpec(memory_space=pltpu.SMEM),
            pl.BlockSpec(memory_space=pltpu.SMEM),
            pl.BlockSpec((npad, 3), lambda t: (0, 0)),
            pl.BlockSpec((npad, 1), lambda t: (0, 0)),
            pl.BlockSpec((3, TILE_DST), lambda t: (0, t + toff)),
            pl.BlockSpec((1, TILE_DST), lambda t: (0, t + toff)),
        ],
        out_specs=pl.BlockSpec((K_NN, TILE_DST), lambda t: (0, t)),
        out_shape=jax.ShapeDtypeStruct((K_NN, nt * TILE_DST), jnp.int32),
        scratch_shapes=[pltpu.VMEM((npad, TILE_DST), jnp.int32)],
    )(lo_blk, hi_blk, pcand, bcand, pdst, bdst)


GATHER_WIN = 128


def _sc_gather(table, idx_flat):
    n_idx = idx_flat.shape[1]
    dim = table.shape[1]
    mesh = plsc.VectorSubcoreMesh(core_axis_name="core",
                                  subcore_axis_name="subcore")

    @functools.partial(
        pl.kernel,
        out_type=jax.ShapeDtypeStruct((n_idx, dim), jnp.float32),
        mesh=mesh)
    def gather_kernel(x_hbm, i_hbm, o_hbm):
        def body(i_vmem, o_vmem):
            pltpu.sync_copy(x_hbm.at[i_vmem.at[0]], o_vmem)

        pltpu.emit_pipeline(
            body,
            grid=(n_idx // GATHER_WIN,),
            in_specs=[pl.BlockSpec((1, GATHER_WIN), lambda i: (0, i))],
            out_specs=[pl.BlockSpec((GATHER_WIN, dim), lambda i: (i, 0))],
            core_axis_name=("core", "subcore"),
            dimension_semantics=(pltpu.PARALLEL,),
        )(i_hbm, o_hbm)

    return gather_kernel(table, idx_flat)


TILE_EDGE = 256


def _edge_mlp(xi, xj, din, w1_ref, b1_ref, w2_ref, b2_ref, w3_ref, b3_ref):
    ne = TILE_EDGE * K_NN
    xi = xi[:, :din]
    xj = xj[:, :, :din].reshape(ne, din)
    xirep = jnp.broadcast_to(xi[None, :, :],
                             (K_NN, TILE_EDGE, din)).reshape(ne, din)
    inp = jnp.concatenate([xirep, xj - xirep], axis=1)
    hv = jnp.maximum(
        jnp.dot(inp, w1_ref[...], preferred_element_type=jnp.float32,
                precision=jax.lax.Precision.DEFAULT) + b1_ref[...], 0.0)
    hv = jnp.maximum(
        jnp.dot(hv, w2_ref[...], preferred_element_type=jnp.float32,
                precision=jax.lax.Precision.DEFAULT) + b2_ref[...], 0.0)
    m = jnp.dot(hv, w3_ref[...], preferred_element_type=jnp.float32,
                precision=jax.lax.Precision.DEFAULT) + b3_ref[...]
    dout = m.shape[1]
    mm = jnp.max(m.reshape(K_NN, TILE_EDGE, dout), axis=0)
    return jnp.maximum(mm, 0.0)


def _edge0_body(x_ref, g_ref, w1_ref, b1_ref, w2_ref, b2_ref, w3_ref, b3_ref,
                h_ref, *, din):
    hv = _edge_mlp(x_ref[...], g_ref[...], din,
                   w1_ref, b1_ref, w2_ref, b2_ref, w3_ref, b3_ref)
    pad = h_ref.shape[1] - hv.shape[1]
    h_ref[...] = jnp.concatenate(
        [hv, jnp.zeros((TILE_EDGE, pad), jnp.float32)], axis=1)


def _edge0_pallas(xp, g, w1, b1, w2, b2, w3, b3, eoff, nrows):
    din = xp.shape[1]
    h = w2.shape[0]
    dout = w3.shape[1]
    body = functools.partial(_edge0_body, din=din)
    return pl.pallas_call(
        body,
        grid=(nrows // TILE_EDGE,),
        in_specs=[
            pl.BlockSpec((TILE_EDGE, din), lambda t: (t + eoff, 0)),
            pl.BlockSpec((K_NN, TILE_EDGE, 128), lambda t: (0, t, 0)),
            pl.BlockSpec((2 * din, h), lambda t: (0, 0)),
            pl.BlockSpec((1, h), lambda t: (0, 0)),
            pl.BlockSpec((h, h), lambda t: (0, 0)),
            pl.BlockSpec((1, h), lambda t: (0, 0)),
            pl.BlockSpec((h, dout), lambda t: (0, 0)),
            pl.BlockSpec((1, dout), lambda t: (0, 0)),
        ],
        out_specs=pl.BlockSpec((TILE_EDGE, 128), lambda t: (t, 0)),
        out_shape=jax.ShapeDtypeStruct((nrows, 128), jnp.float32),
    )(xp, g, w1, b1, w2, b2, w3, b3)


def _edge1a_body(x_ref, g_ref, w1_ref, b1_ref, w2_ref, b2_ref, w3_ref,
                 b3_ref, bcol_ref, psum_ref, pcnt_ref, sum_ref, cnt_ref,
                 *, din):
    t = pl.program_id(0)

    @pl.when(t == 0)
    def _():
        sum_ref[...] = jnp.zeros_like(sum_ref)
        cnt_ref[...] = jnp.zeros_like(cnt_ref)

    hv = _edge_mlp(x_ref[...], g_ref[...], din,
                   w1_ref, b1_ref, w2_ref, b2_ref, w3_ref, b3_ref)

    bc = bcol_ref[...]
    g = jax.lax.broadcasted_iota(jnp.int32, (N_GRAPHS, 1), 0)
    oh = (bc == g).astype(jnp.float32)
    sum_ref[...] += jnp.dot(oh, hv, preferred_element_type=jnp.float32,
                            precision=jax.lax.Precision.HIGHEST)
    cnt_ref[...] += jnp.sum(oh, axis=1, keepdims=True)

    @pl.when(t == pl.num_programs(0) - 1)
    def _():
        psum_ref[...] = sum_ref[...]
        pcnt_ref[...] = cnt_ref[...]


def _edge1b_body(x_ref, g_ref, w1_ref, b1_ref, w2_ref, b2_ref, w3_ref,
                 b3_ref, bcol_ref, psum_ref, pcnt_ref,
                 l1_ref, bl1_ref, l2_ref, bl2_ref, l3_ref, bl3_ref,
                 o_ref, sum_ref, cnt_ref, *, din):
    t = pl.program_id(0)

    @pl.when(t == 0)
    def _():
        sum_ref[...] = psum_ref[...]
        cnt_ref[...] = pcnt_ref[...]

    hv = _edge_mlp(x_ref[...], g_ref[...], din,
                   w1_ref, b1_ref, w2_ref, b2_ref, w3_ref, b3_ref)

    bc = bcol_ref[...]
    g = jax.lax.broadcasted_iota(jnp.int32, (N_GRAPHS, 1), 0)
    oh = (bc == g).astype(jnp.float32)
    sum_ref[...] += jnp.dot(oh, hv, preferred_element_type=jnp.float32,
                            precision=jax.lax.Precision.HIGHEST)
    cnt_ref[...] += jnp.sum(oh, axis=1, keepdims=True)

    @pl.when(t == pl.num_programs(0) - 1)
    def _():
        pooled = sum_ref[...] / jnp.maximum(cnt_ref[...], 1.0)
        o1 = jnp.maximum(
            jnp.dot(pooled, l1_ref[...], preferred_element_type=jnp.float32,
                    precision=jax.lax.Precision.DEFAULT) + bl1_ref[...], 0.0)
        o2 = jnp.maximum(
            jnp.dot(o1, l2_ref[...], preferred_element_type=jnp.float32,
                    precision=jax.lax.Precision.DEFAULT) + bl2_ref[...], 0.0)
        o_ref[...] = jnp.dot(o2, l3_ref[...],
                             preferred_element_type=jnp.float32,
                             precision=jax.lax.Precision.DEFAULT) + bl3_ref[...]


def _edge1_specs(din, h, dout, eoff):
    return [
        pl.BlockSpec((TILE_EDGE, 128), lambda t: (t + eoff, 0)),
        pl.BlockSpec((K_NN, TILE_EDGE, 128), lambda t: (0, t, 0)),
        pl.BlockSpec((2 * din, h), lambda t: (0, 0)),
        pl.BlockSpec((1, h), lambda t: (0, 0)),
        pl.BlockSpec((h, h), lambda t: (0, 0)),
        pl.BlockSpec((1, h), lambda t: (0, 0)),
        pl.BlockSpec((h, dout), lambda t: (0, 0)),
        pl.BlockSpec((1, dout), lambda t: (0, 0)),
        pl.BlockSpec((1, TILE_EDGE), lambda t: (0, t + eoff)),
    ]


def _edge1a_pallas(hp, g, w1, b1, w2, b2, w3, b3, bcol, eoff, nrows, din):
    h = w2.shape[0]
    dout = w3.shape[1]
    body = functools.partial(_edge1a_body, din=din)
    return pl.pallas_call(
        body,
        grid=(nrows // TILE_EDGE,),
        in_specs=_edge1_specs(din, h, dout, eoff),
        out_specs=[
            pl.BlockSpec((N_GRAPHS, dout), lambda t: (0, 0)),
            pl.BlockSpec((N_GRAPHS, 1), lambda t: (0, 0)),
        ],
        out_shape=[
            jax.ShapeDtypeStruct((N_GRAPHS, dout), jnp.float32),
            jax.ShapeDtypeStruct((N_GRAPHS, 1), jnp.float32),
        ],
        scratch_shapes=[pltpu.VMEM((N_GRAPHS, dout), jnp.float32),
                        pltpu.VMEM((N_GRAPHS, 1), jnp.float32)],
    )(hp, g, w1, b1, w2, b2, w3, b3, bcol)


def _edge1b_pallas(hp, g, w1, b1, w2, b2, w3, b3, bcol, psum, pcnt,
                   l1, bl1, l2, bl2, l3, bl3, eoff, nrows, din):
    h = w2.shape[0]
    dout = w3.shape[1]
    body = functools.partial(_edge1b_body, din=din)
    specs = _edge1_specs(din, h, dout, eoff) + [
        pl.BlockSpec((N_GRAPHS, dout), lambda t: (0, 0)),
        pl.BlockSpec((N_GRAPHS, 1), lambda t: (0, 0)),
        pl.BlockSpec((dout, dout), lambda t: (0, 0)),
        pl.BlockSpec((1, dout), lambda t: (0, 0)),
        pl.BlockSpec((dout, dout), lambda t: (0, 0)),
        pl.BlockSpec((1, dout), lambda t: (0, 0)),
        pl.BlockSpec((dout, 1), lambda t: (0, 0)),
        pl.BlockSpec((1, 1), lambda t: (0, 0)),
    ]
    return pl.pallas_call(
        body,
        grid=(nrows // TILE_EDGE,),
        in_specs=specs,
        out_specs=pl.BlockSpec((N_GRAPHS, 1), lambda t: (0, 0)),
        out_shape=jax.ShapeDtypeStruct((N_GRAPHS, 1), jnp.float32),
        scratch_shapes=[pltpu.VMEM((N_GRAPHS, dout), jnp.float32),
                        pltpu.VMEM((N_GRAPHS, 1), jnp.float32)],
    )(hp, g, w1, b1, w2, b2, w3, b3, bcol, psum, pcnt,
      l1, bl1, l2, bl2, l3, bl3)


def kernel(x, batch, W1_0, b1_0, W2_0, b2_0, W3_0, b3_0,
           W1_1, b1_1, W2_1, b2_1, W3_1, b3_1, L1, bl1, L2, bl2, L3, bl3):
    n, dfeat = x.shape
    npad = ((n + TILE_EDGE - 1) // TILE_EDGE) * TILE_EDGE
    lat = W3_0.shape[1]
    batch = batch.astype(jnp.int32)

    pos = x[:, :3]
    pcand = jnp.concatenate(
        [pos, jnp.zeros((npad - n, 3), jnp.float32)], axis=0)
    pdst = pcand.T
    bcand = jnp.concatenate(
        [batch, jnp.full((npad - n,), -1, jnp.int32)])[:, None]
    brow = jnp.concatenate(
        [batch, jnp.full((npad - n,), N_GRAPHS - 1, jnp.int32)])
    bdst = brow[None, :]
    bcol = bcand[:, 0][None, :]

    seg = jnp.searchsorted(
        batch, jnp.arange(N_GRAPHS + 1, dtype=jnp.int32),
        side="left").astype(jnp.int32)
    sizes = seg[1:] - seg[:-1]
    nt = npad // TILE_DST
    tidx = jnp.arange(nt, dtype=jnp.int32)
    bfirst = brow[tidx * TILE_DST]
    blast = brow[tidx * TILE_DST + TILE_DST - 1]
    gidx = jnp.arange(N_GRAPHS, dtype=jnp.int32)
    inrange = (gidx[None, :] >= bfirst[:, None]) & \
              (gidx[None, :] <= blast[:, None])
    minsz = jnp.min(jnp.where(inrange, sizes[None, :], n + 1), axis=1)
    degen = minsz < K_NN + 1
    lo = jnp.where(degen, 0, seg[bfirst])
    hi = jnp.where(degen, n, seg[blast + 1])
    lo_blk = lo // BLK_CAND
    hi_blk = (hi + BLK_CAND - 1) // BLK_CAND

    xp = jnp.concatenate(
        [x, jnp.zeros((npad - n, dfeat), jnp.float32)], axis=0)

    half = npad // 2
    nt_h = nt // 2
    et_h = half // TILE_EDGE

    nbr_a = _knn_pallas(pcand, bcand, pdst, bdst,
                        lo_blk[:nt_h], hi_blk[:nt_h], npad, 0, nt_h)
    nbr_b = _knn_pallas(pcand, bcand, pdst, bdst,
                        lo_blk[nt_h:], hi_blk[nt_h:], npad, nt_h, nt_h)

    g0a = _sc_gather(xp, nbr_a.reshape(1, half * K_NN)).reshape(
        K_NN, half, 128)
    g0b = _sc_gather(xp, nbr_b.reshape(1, half * K_NN)).reshape(
        K_NN, half, 128)
    w10 = (W1_0, b1_0[None, :], W2_0, b2_0[None, :], W3_0, b3_0[None, :])
    h1a = _edge0_pallas(xp, g0a, *w10, 0, half)
    h1b = _edge0_pallas(xp, g0b, *w10, et_h, half)
    h1 = jnp.concatenate([h1a, h1b], axis=0)

    g1a = _sc_gather(h1, nbr_a.reshape(1, half * K_NN)).reshape(
        K_NN, half, 128)
    g1b = _sc_gather(h1, nbr_b.reshape(1, half * K_NN)).reshape(
        K_NN, half, 128)
    w11 = (W1_1, b1_1[None, :], W2_1, b2_1[None, :], W3_1, b3_1[None, :])
    psum, pcnt = _edge1a_pallas(h1, g1a, *w11, bcol, 0, half, lat)
    return _edge1b_pallas(h1, g1b, *w11, bcol, psum, pcnt,
                          L1, bl1[None, :], L2, bl2[None, :],
                          L3, bl3.reshape(1, 1), et_h, half, lat)


# speedup vs baseline: 1.0133x; 1.0133x over previous
import functools

import jax
import jax.numpy as jnp
from jax.experimental import pallas as pl
from jax.experimental.pallas import tpu as pltpu
from jax.experimental.pallas import tpu_sc as plsc

K_NN = 16
N_GRAPHS = 16

INF_KEY = 0x7F800000
DEAD_KEY = 0x7FFFFFFF


TILE_DST = 128
BLK_CAND = 320


def _knn_body(lo_ref, hi_ref, pcand_ref, bcand_ref, pdst_ref, bdst_ref,
              nbr_ref, k_ref, *, toff):
    t = pl.program_id(0)
    lo_b = lo_ref[t]
    hi_b = hi_ref[t]

    pd0 = pdst_ref[0:1, :]
    pd1 = pdst_ref[1:2, :]
    pd2 = pdst_ref[2:3, :]
    bd = bdst_ref[0:1, :]
    did = (t + toff) * TILE_DST + jax.lax.broadcasted_iota(
        jnp.int32, (1, TILE_DST), 1)

    big = jnp.int32(2 ** 30)

    def _top2(kb, cid, carry):
        r1m, r1i, r2m, r2i = carry
        m1 = jnp.min(kb, axis=0, keepdims=True)
        i1 = jnp.min(jnp.where(kb == m1, cid, big), axis=0, keepdims=True)
        kb2 = jnp.where(cid == i1, DEAD_KEY, kb)
        m2 = jnp.min(kb2, axis=0, keepdims=True)
        i2 = jnp.min(jnp.where(kb2 == m2, cid, big), axis=0, keepdims=True)
        lt = lambda am, ai, bm, bi: (am < bm) | ((am == bm) & (ai < bi))
        c = lt(m1, i1, r1m, r1i)
        n1m, n1i = jnp.where(c, m1, r1m), jnp.where(c, i1, r1i)
        h1m, h1i = jnp.where(c, r1m, m1), jnp.where(c, r1i, i1)
        d = lt(m2, i2, r2m, r2i)
        l2m, l2i = jnp.where(d, m2, r2m), jnp.where(d, i2, r2i)
        e = lt(h1m, h1i, l2m, l2i)
        n2m, n2i = jnp.where(e, h1m, l2m), jnp.where(e, h1i, l2i)
        return (n1m, n1i, n2m, n2i)

    init2 = (jnp.full((1, TILE_DST), DEAD_KEY, jnp.int32),
             jnp.full((1, TILE_DST), big, jnp.int32),
             jnp.full((1, TILE_DST), DEAD_KEY, jnp.int32),
             jnp.full((1, TILE_DST), big, jnp.int32))

    def dist_block(b, carry):
        base = b * BLK_CAND
        sl = pl.ds(base, BLK_CAND)
        dx0 = pcand_ref[sl, 0:1] - pd0
        dx1 = pcand_ref[sl, 1:2] - pd1
        dx2 = pcand_ref[sl, 2:3] - pd2
        d = dx0 * dx0 + dx1 * dx1 + dx2 * dx2
        bc = bcand_ref[sl, 0:1]
        cid = base + jax.lax.broadcasted_iota(jnp.int32, (BLK_CAND, 1), 0)
        mask = (bc != bd) | (cid == did)
        key = jax.lax.bitcast_convert_type(d, jnp.int32)
        kv = jnp.where(mask, INF_KEY, key)
        k_ref[sl, :] = kv
        return _top2(kv, cid, carry)

    carry = jax.lax.fori_loop(lo_b, hi_b, dist_block, init2)
    picks = [carry[1], carry[3]]
    li1, li2 = carry[1], carry[3]

    for p in range(1, K_NN // 2):
        last = p == K_NN // 2 - 1

        def scan_block(b, carry, li1=li1, li2=li2, last=last):
            base = b * BLK_CAND
            sl = pl.ds(base, BLK_CAND)
            cid = base + jax.lax.broadcasted_iota(
                jnp.int32, (BLK_CAND, 1), 0)
            kb = k_ref[sl, :]
            kb = jnp.where((cid == li1) | (cid == li2), DEAD_KEY, kb)
            if not last:
                k_ref[sl, :] = kb
            return _top2(kb, cid, carry)

        carry = jax.lax.fori_loop(lo_b, hi_b, scan_block, init2)
        picks += [carry[1], carry[3]]
        li1, li2 = carry[1], carry[3]

    nbr_ref[...] = jnp.concatenate(picks, axis=0)


def _knn_pallas(pcand, bcand, pdst, bdst, lo_blk, hi_blk, npad, toff, nt):
    body = functools.partial(_knn_body, toff=toff)
    return pl.pallas_call(
        body,
        grid=(nt,),
        in_specs=[
            pl.BlockSpec(memory_space=pltpu.SMEM),
            pl.BlockSpec(memory_space=pltpu.SMEM),
            pl.BlockSpec((npad, 3), lambda t: (0, 0)),
            pl.BlockSpec((npad, 1), lambda t: (0, 0)),
            pl.BlockSpec((3, TILE_DST), lambda t: (0, t + toff)),
            pl.BlockSpec((1, TILE_DST), lambda t: (0, t + toff)),
        ],
        out_specs=pl.BlockSpec((K_NN, TILE_DST), lambda t: (0, t)),
        out_shape=jax.ShapeDtypeStruct((K_NN, nt * TILE_DST), jnp.int32),
        scratch_shapes=[pltpu.VMEM((npad, TILE_DST), jnp.int32)],
    )(lo_blk, hi_blk, pcand, bcand, pdst, bdst)


GATHER_WIN = 128


def _sc_gather(table, idx_flat):
    n_idx = idx_flat.shape[1]
    dim = table.shape[1]
    mesh = plsc.VectorSubcoreMesh(core_axis_name="core",
                                  subcore_axis_name="subcore")

    @functools.partial(
        pl.kernel,
        out_type=jax.ShapeDtypeStruct((n_idx, dim), jnp.float32),
        mesh=mesh)
    def gather_kernel(x_hbm, i_hbm, o_hbm):
        def body(i_vmem, o_vmem):
            pltpu.sync_copy(x_hbm.at[i_vmem.at[0]], o_vmem)

        pltpu.emit_pipeline(
            body,
            grid=(n_idx // GATHER_WIN,),
            in_specs=[pl.BlockSpec((1, GATHER_WIN), lambda i: (0, i))],
            out_specs=[pl.BlockSpec((GATHER_WIN, dim), lambda i: (i, 0))],
            core_axis_name=("core", "subcore"),
            dimension_semantics=(pltpu.PARALLEL,),
        )(i_hbm, o_hbm)

    return gather_kernel(table, idx_flat)


TILE_EDGE = 512


def _edge_mlp(xi, xj, din, w1_ref, b1_ref, w2_ref, b2_ref, w3_ref, b3_ref):
    ne = TILE_EDGE * K_NN
    xi = xi[:, :din]
    xj = xj[:, :, :din].reshape(ne, din)
    xirep = jnp.broadcast_to(xi[None, :, :],
                             (K_NN, TILE_EDGE, din)).reshape(ne, din)
    inp = jnp.concatenate([xirep, xj - xirep], axis=1)
    hv = jnp.maximum(
        jnp.dot(inp, w1_ref[...], preferred_element_type=jnp.float32,
                precision=jax.lax.Precision.DEFAULT) + b1_ref[...], 0.0)
    hv = jnp.maximum(
        jnp.dot(hv, w2_ref[...], preferred_element_type=jnp.float32,
                precision=jax.lax.Precision.DEFAULT) + b2_ref[...], 0.0)
    m = jnp.dot(hv, w3_ref[...], preferred_element_type=jnp.float32,
                precision=jax.lax.Precision.DEFAULT) + b3_ref[...]
    dout = m.shape[1]
    mm = jnp.max(m.reshape(K_NN, TILE_EDGE, dout), axis=0)
    return jnp.maximum(mm, 0.0)


def _edge0_body(x_ref, g_ref, w1_ref, b1_ref, w2_ref, b2_ref, w3_ref, b3_ref,
                h_ref, *, din):
    hv = _edge_mlp(x_ref[...], g_ref[...], din,
                   w1_ref, b1_ref, w2_ref, b2_ref, w3_ref, b3_ref)
    pad = h_ref.shape[1] - hv.shape[1]
    h_ref[...] = jnp.concatenate(
        [hv, jnp.zeros((TILE_EDGE, pad), jnp.float32)], axis=1)


def _edge0_pallas(xp, g, w1, b1, w2, b2, w3, b3, eoff, nrows):
    din = xp.shape[1]
    h = w2.shape[0]
    dout = w3.shape[1]
    body = functools.partial(_edge0_body, din=din)
    return pl.pallas_call(
        body,
        grid=(nrows // TILE_EDGE,),
        in_specs=[
            pl.BlockSpec((TILE_EDGE, din), lambda t: (t + eoff, 0)),
            pl.BlockSpec((K_NN, TILE_EDGE, 128), lambda t: (0, t, 0)),
            pl.BlockSpec((2 * din, h), lambda t: (0, 0)),
            pl.BlockSpec((1, h), lambda t: (0, 0)),
            pl.BlockSpec((h, h), lambda t: (0, 0)),
            pl.BlockSpec((1, h), lambda t: (0, 0)),
            pl.BlockSpec((h, dout), lambda t: (0, 0)),
            pl.BlockSpec((1, dout), lambda t: (0, 0)),
        ],
        out_specs=pl.BlockSpec((TILE_EDGE, 128), lambda t: (t, 0)),
        out_shape=jax.ShapeDtypeStruct((nrows, 128), jnp.float32),
    )(xp, g, w1, b1, w2, b2, w3, b3)


def _edge1a_body(x_ref, g_ref, w1_ref, b1_ref, w2_ref, b2_ref, w3_ref,
                 b3_ref, bcol_ref, psum_ref, pcnt_ref, sum_ref, cnt_ref,
                 *, din):
    t = pl.program_id(0)

    @pl.when(t == 0)
    def _():
        sum_ref[...] = jnp.zeros_like(sum_ref)
        cnt_ref[...] = jnp.zeros_like(cnt_ref)

    hv = _edge_mlp(x_ref[...], g_ref[...], din,
                   w1_ref, b1_ref, w2_ref, b2_ref, w3_ref, b3_ref)

    bc = bcol_ref[...]
    g = jax.lax.broadcasted_iota(jnp.int32, (N_GRAPHS, 1), 0)
    oh = (bc == g).astype(jnp.float32)
    sum_ref[...] += jnp.dot(oh, hv, preferred_element_type=jnp.float32,
                            precision=jax.lax.Precision.HIGHEST)
    cnt_ref[...] += jnp.sum(oh, axis=1, keepdims=True)

    @pl.when(t == pl.num_programs(0) - 1)
    def _():
        psum_ref[...] = sum_ref[...]
        pcnt_ref[...] = cnt_ref[...]


def _edge1b_body(x_ref, g_ref, w1_ref, b1_ref, w2_ref, b2_ref, w3_ref,
                 b3_ref, bcol_ref, psum_ref, pcnt_ref,
                 l1_ref, bl1_ref, l2_ref, bl2_ref, l3_ref, bl3_ref,
                 o_ref, sum_ref, cnt_ref, *, din):
    t = pl.program_id(0)

    @pl.when(t == 0)
    def _():
        sum_ref[...] = psum_ref[...]
        cnt_ref[...] = pcnt_ref[...]

    hv = _edge_mlp(x_ref[...], g_ref[...], din,
                   w1_ref, b1_ref, w2_ref, b2_ref, w3_ref, b3_ref)

    bc = bcol_ref[...]
    g = jax.lax.broadcasted_iota(jnp.int32, (N_GRAPHS, 1), 0)
    oh = (bc == g).astype(jnp.float32)
    sum_ref[...] += jnp.dot(oh, hv, preferred_element_type=jnp.float32,
                            precision=jax.lax.Precision.HIGHEST)
    cnt_ref[...] += jnp.sum(oh, axis=1, keepdims=True)

    @pl.when(t == pl.num_programs(0) - 1)
    def _():
        pooled = sum_ref[...] / jnp.maximum(cnt_ref[...], 1.0)
        o1 = jnp.maximum(
            jnp.dot(pooled, l1_ref[...], preferred_element_type=jnp.float32,
                    precision=jax.lax.Precision.DEFAULT) + bl1_ref[...], 0.0)
        o2 = jnp.maximum(
            jnp.dot(o1, l2_ref[...], preferred_element_type=jnp.float32,
                    precision=jax.lax.Precision.DEFAULT) + bl2_ref[...], 0.0)
        o_ref[...] = jnp.dot(o2, l3_ref[...],
                             preferred_element_type=jnp.float32,
                             precision=jax.lax.Precision.DEFAULT) + bl3_ref[...]


def _edge1_specs(din, h, dout, eoff):
    return [
        pl.BlockSpec((TILE_EDGE, 128), lambda t: (t + eoff, 0)),
        pl.BlockSpec((K_NN, TILE_EDGE, 128), lambda t: (0, t, 0)),
        pl.BlockSpec((2 * din, h), lambda t: (0, 0)),
        pl.BlockSpec((1, h), lambda t: (0, 0)),
        pl.BlockSpec((h, h), lambda t: (0, 0)),
        pl.BlockSpec((1, h), lambda t: (0, 0)),
        pl.BlockSpec((h, dout), lambda t: (0, 0)),
        pl.BlockSpec((1, dout), lambda t: (0, 0)),
        pl.BlockSpec((1, TILE_EDGE), lambda t: (0, t + eoff)),
    ]


def _edge1a_pallas(hp, g, w1, b1, w2, b2, w3, b3, bcol, eoff, nrows, din):
    h = w2.shape[0]
    dout = w3.shape[1]
    body = functools.partial(_edge1a_body, din=din)
    return pl.pallas_call(
        body,
        grid=(nrows // TILE_EDGE,),
        in_specs=_edge1_specs(din, h, dout, eoff),
        out_specs=[
            pl.BlockSpec((N_GRAPHS, dout), lambda t: (0, 0)),
            pl.BlockSpec((N_GRAPHS, 1), lambda t: (0, 0)),
        ],
        out_shape=[
            jax.ShapeDtypeStruct((N_GRAPHS, dout), jnp.float32),
            jax.ShapeDtypeStruct((N_GRAPHS, 1), jnp.float32),
        ],
        scratch_shapes=[pltpu.VMEM((N_GRAPHS, dout), jnp.float32),
                        pltpu.VMEM((N_GRAPHS, 1), jnp.float32)],
    )(hp, g, w1, b1, w2, b2, w3, b3, bcol)


def _edge1b_pallas(hp, g, w1, b1, w2, b2, w3, b3, bcol, psum, pcnt,
                   l1, bl1, l2, bl2, l3, bl3, eoff, nrows, din):
    h = w2.shape[0]
    dout = w3.shape[1]
    body = functools.partial(_edge1b_body, din=din)
    specs = _edge1_specs(din, h, dout, eoff) + [
        pl.BlockSpec((N_GRAPHS, dout), lambda t: (0, 0)),
        pl.BlockSpec((N_GRAPHS, 1), lambda t: (0, 0)),
        pl.BlockSpec((dout, dout), lambda t: (0, 0)),
        pl.BlockSpec((1, dout), lambda t: (0, 0)),
        pl.BlockSpec((dout, dout), lambda t: (0, 0)),
        pl.BlockSpec((1, dout), lambda t: (0, 0)),
        pl.BlockSpec((dout, 1), lambda t: (0, 0)),
        pl.BlockSpec((1, 1), lambda t: (0, 0)),
    ]
    return pl.pallas_call(
        body,
        grid=(nrows // TILE_EDGE,),
        in_specs=specs,
        out_specs=pl.BlockSpec((N_GRAPHS, 1), lambda t: (0, 0)),
        out_shape=jax.ShapeDtypeStruct((N_GRAPHS, 1), jnp.float32),
        scratch_shapes=[pltpu.VMEM((N_GRAPHS, dout), jnp.float32),
                        pltpu.VMEM((N_GRAPHS, 1), jnp.float32)],
    )(hp, g, w1, b1, w2, b2, w3, b3, bcol, psum, pcnt,
      l1, bl1, l2, bl2, l3, bl3)


def kernel(x, batch, W1_0, b1_0, W2_0, b2_0, W3_0, b3_0,
           W1_1, b1_1, W2_1, b2_1, W3_1, b3_1, L1, bl1, L2, bl2, L3, bl3):
    n, dfeat = x.shape
    npad = ((n + TILE_EDGE - 1) // TILE_EDGE) * TILE_EDGE
    lat = W3_0.shape[1]
    batch = batch.astype(jnp.int32)

    pos = x[:, :3]
    pcand = jnp.concatenate(
        [pos, jnp.zeros((npad - n, 3), jnp.float32)], axis=0)
    pdst = pcand.T
    bcand = jnp.concatenate(
        [batch, jnp.full((npad - n,), -1, jnp.int32)])[:, None]
    brow = jnp.concatenate(
        [batch, jnp.full((npad - n,), N_GRAPHS - 1, jnp.int32)])
    bdst = brow[None, :]
    bcol = bcand[:, 0][None, :]

    seg = jnp.searchsorted(
        batch, jnp.arange(N_GRAPHS + 1, dtype=jnp.int32),
        side="left").astype(jnp.int32)
    sizes = seg[1:] - seg[:-1]
    nt = npad // TILE_DST
    tidx = jnp.arange(nt, dtype=jnp.int32)
    bfirst = brow[tidx * TILE_DST]
    blast = brow[tidx * TILE_DST + TILE_DST - 1]
    gidx = jnp.arange(N_GRAPHS, dtype=jnp.int32)
    inrange = (gidx[None, :] >= bfirst[:, None]) & \
              (gidx[None, :] <= blast[:, None])
    minsz = jnp.min(jnp.where(inrange, sizes[None, :], n + 1), axis=1)
    degen = minsz < K_NN + 1
    lo = jnp.where(degen, 0, seg[bfirst])
    hi = jnp.where(degen, n, seg[blast + 1])
    lo_blk = lo // BLK_CAND
    hi_blk = (hi + BLK_CAND - 1) // BLK_CAND

    xp = jnp.concatenate(
        [x, jnp.zeros((npad - n, dfeat), jnp.float32)], axis=0)

    half = npad // 2
    nt_h = nt // 2
    et_h = half // TILE_EDGE

    nbr_a = _knn_pallas(pcand, bcand, pdst, bdst,
                        lo_blk[:nt_h], hi_blk[:nt_h], npad, 0, nt_h)
    nbr_b = _knn_pallas(pcand, bcand, pdst, bdst,
                        lo_blk[nt_h:], hi_blk[nt_h:], npad, nt_h, nt_h)

    g0a = _sc_gather(xp, nbr_a.reshape(1, half * K_NN)).reshape(
        K_NN, half, 128)
    g0b = _sc_gather(xp, nbr_b.reshape(1, half * K_NN)).reshape(
        K_NN, half, 128)
    w10 = (W1_0, b1_0[None, :], W2_0, b2_0[None, :], W3_0, b3_0[None, :])
    h1a = _edge0_pallas(xp, g0a, *w10, 0, half)
    h1b = _edge0_pallas(xp, g0b, *w10, et_h, half)
    h1 = jnp.concatenate([h1a, h1b], axis=0)

    g1a = _sc_gather(h1, nbr_a.reshape(1, half * K_NN)).reshape(
        K_NN, half, 128)
    g1b = _sc_gather(h1, nbr_b.reshape(1, half * K_NN)).reshape(
        K_NN, half, 128)
    w11 = (W1_1, b1_1[None, :], W2_1, b2_1[None, :], W3_1, b3_1[None, :])
    psum, pcnt = _edge1a_pallas(h1, g1a, *w11, bcol, 0, half, lat)
    return _edge1b_pallas(h1, g1b, *w11, bcol, psum, pcnt,
                          L1, bl1[None, :], L2, bl2[None, :],
                          L3, bl3.reshape(1, 1), et_h, half, lat)
